# Initial kernel scaffold; baseline (speedup 1.0000x reference)
import jax
import jax.numpy as jnp
from jax.experimental import pallas as pl


def kernel(x_flat, W):
    raise NotImplementedError("write your pallas kernel here")


# speedup vs baseline: 11.4780x; 11.4780x over previous
import functools

import jax
import jax.numpy as jnp
from jax import lax
from jax.experimental import pallas as pl
from jax.experimental.pallas import tpu as pltpu
from jax.experimental.pallas import tpu_sc as plsc

N = 8192
D = 256
K = 8192
BN = 512


def _argmin_body(x_ref, w_ref, idx_ref, wb_ref, wsq_ref):
    @pl.when(pl.program_id(0) == 0)
    def _():
        w = w_ref[...]
        wb_ref[...] = w.astype(jnp.bfloat16)
        wsq_ref[...] = jnp.sum(w * w, axis=1)[None, :]

    x = x_ref[...]
    xb = x.astype(jnp.bfloat16)
    s = lax.dot_general(
        xb, wb_ref[...], (((1,), (1,)), ((), ())),
        preferred_element_type=jnp.float32,
    )
    xsq = jnp.sum(x * x, axis=1, keepdims=True)
    d = (xsq - 2.0 * s) + wsq_ref[...]
    dmin = jnp.min(d, axis=1, keepdims=True)
    ji = lax.broadcasted_iota(jnp.int32, d.shape, 1)
    idx_ref[...] = jnp.min(jnp.where(d == dmin, ji, K), axis=1)


def _nearest_indices(x_flat, W):
    return pl.pallas_call(
        _argmin_body,
        grid=(N // BN,),
        in_specs=[
            pl.BlockSpec((BN, D), lambda i: (i, 0)),
            pl.BlockSpec((K, D), lambda i: (0, 0)),
        ],
        out_specs=pl.BlockSpec((BN,), lambda i: (i,)),
        out_shape=jax.ShapeDtypeStruct((N,), jnp.int32),
        scratch_shapes=[
            pltpu.VMEM((K, D), jnp.bfloat16),
            pltpu.VMEM((1, K), jnp.float32),
        ],
        compiler_params=pltpu.CompilerParams(
            dimension_semantics=("arbitrary",),
        ),
    )(x_flat, W)


def _gather_rows(W, idx):
    info = plsc.get_sparse_core_info()
    nw = info.num_cores * info.num_subcores
    bpw = N // nw
    mesh = plsc.VectorSubcoreMesh(core_axis_name="c", subcore_axis_name="s")

    @functools.partial(
        pl.kernel,
        mesh=mesh,
        out_type=jax.ShapeDtypeStruct((N, D), jnp.float32),
        scratch_types=[
            pltpu.VMEM((bpw,), jnp.int32),
            pltpu.VMEM((bpw, D), jnp.float32),
            pltpu.SemaphoreType.DMA,
        ],
    )
    def k(w_hbm, idx_hbm, out_hbm, idx_v, rows_v, sem):
        wid = lax.axis_index("s") * info.num_cores + lax.axis_index("c")
        base = wid * bpw
        pltpu.sync_copy(idx_hbm.at[pl.ds(base, bpw)], idx_v)
        pltpu.async_copy(w_hbm.at[idx_v], rows_v, sem).wait()
        pltpu.sync_copy(rows_v, out_hbm.at[pl.ds(base, bpw)])

    return k(W, idx)


def kernel(x_flat, W):
    idx = _nearest_indices(x_flat, W)
    return _gather_rows(W, idx)
